# baseline (speedup 1.0000x reference)
import jax
import jax.numpy as jnp
from jax import lax
from jax.experimental import pallas as pl
from jax.experimental.pallas import tpu as pltpu
from jax.experimental.pallas import tpu_sc as plsc

_L = 2
_B = 80
_BN = 128
_K = 512
_NCH = 13
_SQRT_2_OVER_PI = 0.7978845608028654
_GELU_C = 0.044715


def _gelu_tc(v):
    return 0.5 * v * (1.0 + jnp.tanh(_SQRT_2_OVER_PI * (v + _GELU_C * v * v * v)))


def _edge_c_body(attr_ref, wc_ref, bc_ref, out_ref):
    a = attr_ref[...]
    for l in range(_L):
        for k in range(2):
            j = l * 2 + k
            w = wc_ref[:, j * 128:(j + 1) * 128]
            out_ref[l, k] = (
                jnp.dot(a, w, preferred_element_type=jnp.float32) + bc_ref[j])


def _edge_c(edge_attr, wc, bc, E, D):
    Be = 1000
    return pl.pallas_call(
        _edge_c_body,
        grid=(E // Be,),
        in_specs=[
            pl.BlockSpec((Be, D), lambda i: (i, 0)),
            pl.BlockSpec((D, 4 * D), lambda i: (0, 0)),
            pl.BlockSpec((4, D), lambda i: (0, 0)),
        ],
        out_specs=pl.BlockSpec((_L, 2, Be, D), lambda i: (0, 0, i, 0)),
        out_shape=jax.ShapeDtypeStruct((_L, 2, E, D), jnp.float32),
    )(edge_attr, wc, bc)


def _node_body(x_ref, w_ref, out_ref):
    xb = x_ref[...]
    for t in range(2):
        for k in range(2):
            j = t * 2 + k
            out_ref[t, k] = jnp.dot(
                xb, w_ref[:, j * 128:(j + 1) * 128],
                preferred_element_type=jnp.float32)


def _node_tables(x, wnode, N, D):
    Bn = 1000
    return pl.pallas_call(
        _node_body,
        grid=(N // Bn,),
        in_specs=[
            pl.BlockSpec((Bn, D), lambda i: (i, 0)),
            pl.BlockSpec((D, 4 * D), lambda i: (0, 0)),
        ],
        out_specs=pl.BlockSpec((2, 2, Bn, D), lambda i: (0, 0, i, 0)),
        out_shape=jax.ShapeDtypeStruct((2, 2, N, D), jnp.float32),
    )(x, wnode)


def _sc_build(E, D):
    mesh = plsc.VectorSubcoreMesh(core_axis_name="c", subcore_axis_name="s")
    n_sub = 16
    per_tile = E // n_sub
    n_chunks = per_tile // _B

    scratch = [
        pltpu.VMEM((_B,), jnp.int32),
        pltpu.VMEM((_B,), jnp.int32),
        pltpu.VMEM((_B,), jnp.int32),
        pltpu.VMEM((_B, D), jnp.float32),
        pltpu.VMEM((_B, D), jnp.float32),
        pltpu.VMEM((_B, D), jnp.float32),
        pltpu.SemaphoreType.DMA,
        pltpu.SemaphoreType.DMA,
        pltpu.SemaphoreType.DMA,
    ]

    def body(ip, iq, ic, p2, q2, c2, s_o,
             idx_p, idx_q, idx_c, rp, rq, rc, s1, s2, s3):
        c = lax.axis_index("c")
        s = lax.axis_index("s")
        base = c * E + s * per_tile

        def chunk(g, carry):
            off = base + g * _B
            pltpu.sync_copy(ip.at[pl.ds(off, _B)], idx_p)
            pltpu.sync_copy(iq.at[pl.ds(off, _B)], idx_q)
            pltpu.sync_copy(ic.at[pl.ds(off, _B)], idx_c)
            d1 = pltpu.async_copy(p2.at[idx_p], rp, s1)
            d2 = pltpu.async_copy(q2.at[idx_q], rq, s2)
            d3 = pltpu.async_copy(c2.at[idx_c], rc, s3)
            d1.wait()
            d2.wait()
            d3.wait()

            def row(r, rcarry):
                for c16 in range(D // 16):
                    sl = pl.ds(c16 * 16, 16)
                    rp[r, sl] = rp[r, sl] + rq[r, sl] + rc[r, sl]
                return rcarry
            lax.fori_loop(0, _B, row, 0)

            pltpu.sync_copy(rp, s_o.at[pl.ds(off, _B)])
            return carry
        lax.fori_loop(0, n_chunks, chunk, 0)

    return pl.kernel(
        body,
        out_type=jax.ShapeDtypeStruct((2 * E, D), jnp.float32),
        mesh=mesh,
        scratch_types=scratch)


def _agg_build(E, D, nb):
    nbp = nb + 1

    def body(starts_ref, s_any, ids_any, sums_ref, cnt_ref,
             sbuf, ibuf, sem1, sem2):
        c = pl.program_id(0)
        b = pl.program_id(1)
        k = pl.program_id(2)
        seg0 = starts_ref[c * nbp + b]
        seg1 = starts_ref[c * nbp + b + 1]

        @pl.when(k == 0)
        def _init():
            sums_ref[...] = jnp.zeros_like(sums_ref)
            cnt_ref[...] = jnp.zeros_like(cnt_ref)

        base = (seg0 // _K) * _K + k * _K

        @pl.when(base < seg1)
        def _work():
            row0 = c * E + base
            cp1 = pltpu.make_async_copy(s_any.at[pl.ds(row0, _K)], sbuf, sem1)
            cp2 = pltpu.make_async_copy(ids_any.at[pl.ds(row0, _K)], ibuf, sem2)
            cp1.start()
            cp2.start()
            cp1.wait()
            cp2.wait()
            u = _gelu_tc(sbuf[...])
            local = ibuf[...] - b * _BN
            rid = lax.broadcasted_iota(jnp.int32, (_BN, _K), 0)
            epos = lax.broadcasted_iota(jnp.int32, (_BN, _K), 1) + base
            oh = jnp.where((local[None, :] == rid) & (epos < seg1), 1.0, 0.0)
            sums_ref[0] += jnp.dot(oh, u, preferred_element_type=jnp.float32)
            cnt_ref[0] += jnp.sum(oh, axis=1, keepdims=True)

    def agg(starts, s_pad, ids_pad):
        return pl.pallas_call(
            body,
            grid_spec=pltpu.PrefetchScalarGridSpec(
                num_scalar_prefetch=1,
                grid=(2, nb, _NCH),
                in_specs=[
                    pl.BlockSpec(memory_space=pl.ANY),
                    pl.BlockSpec(memory_space=pl.ANY),
                ],
                out_specs=[
                    pl.BlockSpec((1, _BN, D), lambda c, b, k, sref: (c, b, 0)),
                    pl.BlockSpec((1, _BN, 1), lambda c, b, k, sref: (c, b, 0)),
                ],
                scratch_shapes=[
                    pltpu.VMEM((_K, D), jnp.float32),
                    pltpu.VMEM((_K,), jnp.int32),
                    pltpu.SemaphoreType.DMA,
                    pltpu.SemaphoreType.DMA,
                ],
            ),
            out_shape=[
                jax.ShapeDtypeStruct((2, nb * _BN, D), jnp.float32),
                jax.ShapeDtypeStruct((2, nb * _BN, 1), jnp.float32),
            ],
        )(starts, s_pad, ids_pad)

    return agg


def _merge_body(x_ref, sums_ref, cnt_ref, g_ref, lng_ref, lnb_ref, out_ref):
    xb = x_ref[...]
    fw = sums_ref[0] / jnp.maximum(cnt_ref[0], 1.0)
    bw = sums_ref[1] / jnp.maximum(cnt_ref[1], 1.0)
    h = (jnp.dot(xb, g_ref[0], preferred_element_type=jnp.float32)
         + jnp.dot(fw, g_ref[1], preferred_element_type=jnp.float32)
         + jnp.dot(bw, g_ref[2], preferred_element_type=jnp.float32))
    y = xb + _gelu_tc(h)
    mu = jnp.mean(y, axis=-1, keepdims=True)
    var = jnp.mean((y - mu) ** 2, axis=-1, keepdims=True)
    out_ref[...] = (y - mu) * lax.rsqrt(var + 1e-5) * lng_ref[...] + lnb_ref[...]


def _merge(x, sums, counts, g3, lng, lnb, N, D):
    Bn = 1000
    return pl.pallas_call(
        _merge_body,
        grid=(N // Bn,),
        in_specs=[
            pl.BlockSpec((Bn, D), lambda i: (i, 0)),
            pl.BlockSpec((2, Bn, D), lambda i: (0, i, 0)),
            pl.BlockSpec((2, Bn, 1), lambda i: (0, i, 0)),
            pl.BlockSpec((3, D, D), lambda i: (0, 0, 0)),
            pl.BlockSpec((1, D), lambda i: (0, 0)),
            pl.BlockSpec((1, D), lambda i: (0, 0)),
        ],
        out_specs=pl.BlockSpec((Bn, D), lambda i: (i, 0)),
        out_shape=jax.ShapeDtypeStruct((N, D), jnp.float32),
    )(x, sums, counts, g3, lng, lnb)


def kernel(x, edge_index, edge_attr, fw_Wn, fw_bn, fw_We, fw_be, fw_Wm, fw_bm,
           bw_Wn, bw_bn, bw_We, bw_be, bw_Wm, bw_bm, mg_W, ln_g, ln_b):
    N, D = x.shape
    E = edge_index.shape[1]
    nb = (N + _BN - 1) // _BN

    Wn = (fw_Wn, bw_Wn); bn = (fw_bn, bw_bn)
    We = (fw_We, bw_We); be = (fw_be, bw_be)
    Wm = (fw_Wm, bw_Wm); bm = (fw_bm, bw_bm)

    wc_cols, bc_rows, wnode = [], [], []
    for l in range(_L):
        pcols, qcols = [], []
        for k in range(2):
            m = Wm[k][l]
            wm1, wm2, wm3 = m[:D], m[D:2 * D], m[2 * D:]
            wc_cols.append(We[k][l] @ wm2)
            bc_rows.append(be[k][l] @ wm2 + bn[k][l] @ wm3 + bm[k][l])
            pcols.append(wm1)
            qcols.append(Wn[k][l] @ wm3)
        wnode.append(jnp.concatenate(pcols + qcols, axis=1))
    wc = jnp.concatenate(wc_cols, axis=1)
    bc = jnp.stack(bc_rows)

    c_all = _edge_c(edge_attr, wc, bc, E, D)

    src, dst = edge_index[0], edge_index[1]
    perm0 = jnp.argsort(dst)
    perm1 = jnp.argsort(src)
    d0 = dst[perm0]
    s1 = src[perm1]
    ip = jnp.concatenate([d0, s1 + N])
    iq = jnp.concatenate([src[perm0], dst[perm1] + N])
    ic = jnp.concatenate([perm0, perm1 + E])
    big = jnp.full((_K,), jnp.int32(1 << 30))
    ids_pad = jnp.concatenate([d0, s1, big])
    bounds = jnp.arange(nb + 1, dtype=jnp.int32) * _BN
    starts = jnp.concatenate([
        jnp.searchsorted(d0, bounds).astype(jnp.int32),
        jnp.searchsorted(s1, bounds).astype(jnp.int32),
    ])

    sc = _sc_build(E, D)
    agg = _agg_build(E, D, nb)
    zpad = jnp.zeros((_K, D), jnp.float32)
    counts = None
    for l in range(_L):
        pq = _node_tables(x, wnode[l], N, D)
        p2 = pq[0].reshape(2 * N, D)
        q2 = pq[1].reshape(2 * N, D)
        c2 = c_all[l].reshape(2 * E, D)
        s_sorted = sc(ip, iq, ic, p2, q2, c2)
        s_pad = jnp.concatenate([s_sorted, zpad])
        sums, cnt_l = agg(starts, s_pad, ids_pad)
        if l == 0:
            counts = cnt_l[:, :N]
        sums = sums[:, :N]
        g3 = mg_W[l].reshape(3, D, D)
        x = _merge(x, sums, counts, g3, ln_g[l].reshape(1, D),
                   ln_b[l].reshape(1, D), N, D)
    return x
